# baseline (speedup 1.0000x reference)
import functools

import jax
import jax.numpy as jnp
from jax import lax
from jax.experimental import pallas as pl
from jax.experimental.pallas import tpu as pltpu
from jax.experimental.pallas import tpu_sc as plsc

N, P, D = 4, 4096, 3
L = 16
NW = 32
WPB = NW // N
SPW = P // WPB
NCH = P // L
SB = 16

_mesh = plsc.VectorSubcoreMesh(
    core_axis_name="c", subcore_axis_name="s", num_cores=2, num_subcores=16
)


def _bfly(v, op):
    lane = lax.iota(jnp.int32, L)
    for k in (8, 4, 2, 1):
        v = op(v, v.at[lane ^ k].get(mode="promise_in_bounds"))
    return v[0]


def _chamfer_sc_body(src_hbm, tgt_hbm, srcb_hbm, tgtb_hbm, out_hbm,
                     tgt_v, t2_v, rtg_v, src_v, srcb_v):
    wid = lax.axis_index("s") * 2 + lax.axis_index("c")
    b = wid // WPB
    slot = wid % WPB

    pltpu.sync_copy(tgt_hbm.at[b], tgt_v)
    pltpu.sync_copy(tgtb_hbm.at[b], rtg_v)
    pltpu.sync_copy(src_hbm.at[b, :, pl.ds(slot * SPW, SPW)], src_v)
    pltpu.sync_copy(srcb_hbm.at[b, :, pl.ds(slot * SPW, SPW)], srcb_v)

    def t2_body(j, carry):
        o = j * L
        tx = tgt_v[0, pl.ds(o, L)]
        ty = tgt_v[1, pl.ds(o, L)]
        tz = tgt_v[2, pl.ds(o, L)]
        t2_v[pl.ds(o, L)] = tx * tx + ty * ty + tz * tz
        return carry

    lax.fori_loop(0, NCH, t2_body, jnp.int32(0))

    big = jnp.float32(3.0e38)
    neg2 = jnp.float32(-2.0)

    def blk_body(bi, acc):
        base = bi * SB
        sxv = src_v[0, pl.ds(base, SB)]
        syv = src_v[1, pl.ds(base, SB)]
        szv = src_v[2, pl.ds(base, SB)]
        s2v = sxv * sxv + syv * syv + szv * szv
        axv = neg2 * srcb_v[0, pl.ds(base, SB)]
        ayv = neg2 * srcb_v[1, pl.ds(base, SB)]
        azv = neg2 * srcb_v[2, pl.ds(base, SB)]
        ax = [axv[i] for i in range(SB)]
        ay = [ayv[i] for i in range(SB)]
        az = [azv[i] for i in range(SB)]

        def ch_body(j, ms):
            o = j * L
            tx = rtg_v[0, pl.ds(o, L)]
            ty = rtg_v[1, pl.ds(o, L)]
            tz = rtg_v[2, pl.ds(o, L)]
            t2 = t2_v[pl.ds(o, L)]
            return tuple(
                jnp.minimum(ms[i], t2 + ax[i] * tx + ay[i] * ty + az[i] * tz)
                for i in range(SB)
            )

        ms0 = tuple(jnp.full((L,), big, jnp.float32) for _ in range(SB))
        ms = lax.fori_loop(0, NCH, ch_body, ms0)
        acc = acc + _bfly(s2v, jnp.add)
        for i in range(SB):
            acc = acc + _bfly(ms[i], jnp.minimum)
        return acc

    total = lax.fori_loop(0, SPW // SB, blk_body, jnp.float32(0.0))
    out_hbm_row = out_hbm.at[wid]
    res = jnp.full((L,), total, jnp.float32)

    def write_res(res_v):
        res_v[...] = res
        pltpu.sync_copy(res_v, out_hbm_row)

    pl.run_scoped(write_res, pltpu.VMEM((L,), jnp.float32))


_chamfer_sc = functools.partial(
    pl.kernel,
    out_type=jax.ShapeDtypeStruct((NW, L), jnp.float32),
    mesh=_mesh,
    scratch_types=[
        pltpu.VMEM((D, P), jnp.float32),
        pltpu.VMEM((P,), jnp.float32),
        pltpu.VMEM((D, P), jnp.float32),
        pltpu.VMEM((D, SPW), jnp.float32),
        pltpu.VMEM((D, SPW), jnp.float32),
    ],
)(_chamfer_sc_body)


def _round_bf16(x):
    i = lax.bitcast_convert_type(x, jnp.int32)
    odd = lax.shift_right_logical(i, 16) & jnp.int32(1)
    i = (i + jnp.int32(0x7FFF) + odd) & jnp.int32(~0xFFFF)
    return lax.bitcast_convert_type(i, jnp.float32)


def kernel(source_cloud, target_cloud):
    src_t = jnp.transpose(source_cloud, (0, 2, 1))
    tgt_t = jnp.transpose(target_cloud, (0, 2, 1))
    srcb_t = _round_bf16(src_t)
    tgtb_t = _round_bf16(tgt_t)
    partials = _chamfer_sc(src_t, tgt_t, srcb_t, tgtb_t)
    return jnp.sum(partials[:, 0]) / N


# speedup vs baseline: 1.8210x; 1.8210x over previous
import functools

import jax
import jax.numpy as jnp
from jax import lax
from jax.experimental import pallas as pl
from jax.experimental.pallas import tpu as pltpu
from jax.experimental.pallas import tpu_sc as plsc

N, P, D = 4, 4096, 3
L = 16
NW = 32
WPB = NW // N
SPW = P // WPB
NCH = P // L
SB = 8

_mesh = plsc.VectorSubcoreMesh(
    core_axis_name="c", subcore_axis_name="s", num_cores=2, num_subcores=16
)


def _bfly(v, op):
    lane = lax.iota(jnp.int32, L)
    for k in (8, 4, 2, 1):
        v = op(v, v.at[lane ^ k].get(mode="promise_in_bounds"))
    return v[0]


def _chamfer_sc_body(src_hbm, tgt_hbm, srcb_hbm, tgtb_hbm, out_hbm,
                     tgt_v, t2_v, rtg_v, src_v, srcb_v):
    wid = lax.axis_index("s") * 2 + lax.axis_index("c")
    b = wid // WPB
    slot = wid % WPB

    pltpu.sync_copy(tgt_hbm.at[b], tgt_v)
    pltpu.sync_copy(tgtb_hbm.at[b], rtg_v)
    pltpu.sync_copy(src_hbm.at[b, :, pl.ds(slot * SPW, SPW)], src_v)
    pltpu.sync_copy(srcb_hbm.at[b, :, pl.ds(slot * SPW, SPW)], srcb_v)

    def t2_body(j, carry):
        o = j * L
        tx = tgt_v[0, pl.ds(o, L)]
        ty = tgt_v[1, pl.ds(o, L)]
        tz = tgt_v[2, pl.ds(o, L)]
        t2_v[pl.ds(o, L)] = tx * tx + ty * ty + tz * tz
        return carry

    lax.fori_loop(0, NCH, t2_body, jnp.int32(0))

    big = jnp.float32(3.0e38)
    neg2 = jnp.float32(-2.0)

    def blk_body(bi, acc):
        base = bi * L
        sxv = src_v[0, pl.ds(base, L)]
        syv = src_v[1, pl.ds(base, L)]
        szv = src_v[2, pl.ds(base, L)]
        s2v = sxv * sxv + syv * syv + szv * szv
        axv = neg2 * srcb_v[0, pl.ds(base, L)]
        ayv = neg2 * srcb_v[1, pl.ds(base, L)]
        azv = neg2 * srcb_v[2, pl.ds(base, L)]
        acc = acc + _bfly(s2v, jnp.add)

        for h in range(L // SB):
            ax = [axv[h * SB + i] for i in range(SB)]
            ay = [ayv[h * SB + i] for i in range(SB)]
            az = [azv[h * SB + i] for i in range(SB)]

            def ch_body(j, ms):
                o = j * L
                tx = rtg_v[0, pl.ds(o, L)]
                ty = rtg_v[1, pl.ds(o, L)]
                tz = rtg_v[2, pl.ds(o, L)]
                t2 = t2_v[pl.ds(o, L)]
                return tuple(
                    jnp.minimum(ms[i], t2 + ax[i] * tx + ay[i] * ty + az[i] * tz)
                    for i in range(SB)
                )

            ms0 = tuple(jnp.full((L,), big, jnp.float32) for _ in range(SB))
            ms = lax.fori_loop(0, NCH, ch_body, ms0)
            for i in range(SB):
                acc = acc + _bfly(ms[i], jnp.minimum)
        return acc

    total = lax.fori_loop(0, SPW // L, blk_body, jnp.float32(0.0))
    out_hbm_row = out_hbm.at[wid]
    res = jnp.full((L,), total, jnp.float32)

    def write_res(res_v):
        res_v[...] = res
        pltpu.sync_copy(res_v, out_hbm_row)

    pl.run_scoped(write_res, pltpu.VMEM((L,), jnp.float32))


_chamfer_sc = functools.partial(
    pl.kernel,
    out_type=jax.ShapeDtypeStruct((NW, L), jnp.float32),
    mesh=_mesh,
    scratch_types=[
        pltpu.VMEM((D, P), jnp.float32),
        pltpu.VMEM((P,), jnp.float32),
        pltpu.VMEM((D, P), jnp.float32),
        pltpu.VMEM((D, SPW), jnp.float32),
        pltpu.VMEM((D, SPW), jnp.float32),
    ],
)(_chamfer_sc_body)


def _round_bf16(x):
    i = lax.bitcast_convert_type(x, jnp.int32)
    odd = lax.shift_right_logical(i, 16) & jnp.int32(1)
    i = (i + jnp.int32(0x7FFF) + odd) & jnp.int32(~0xFFFF)
    return lax.bitcast_convert_type(i, jnp.float32)


def kernel(source_cloud, target_cloud):
    src_t = jnp.transpose(source_cloud, (0, 2, 1))
    tgt_t = jnp.transpose(target_cloud, (0, 2, 1))
    srcb_t = _round_bf16(src_t)
    tgtb_t = _round_bf16(tgt_t)
    partials = _chamfer_sc(src_t, tgt_t, srcb_t, tgtb_t)
    return jnp.sum(partials[:, 0]) / N


# speedup vs baseline: 4.2603x; 2.3396x over previous
import functools

import jax
import jax.numpy as jnp
from jax import lax
from jax.experimental import pallas as pl
from jax.experimental.pallas import tpu as pltpu
from jax.experimental.pallas import tpu_sc as plsc

N, P, D = 4, 4096, 3
L = 16
NW = 32
WPB = NW // N
NCH = P // L
SC_P = 512
TC_P = P - SC_P
TB = 512
SPW = SC_P // WPB
assert SC_P % (WPB * L) == 0 and TC_P % TB == 0
SB = 8

_mesh = plsc.VectorSubcoreMesh(
    core_axis_name="c", subcore_axis_name="s", num_cores=2, num_subcores=16
)


def _bfly(v, op):
    lane = lax.iota(jnp.int32, L)
    for k in (8, 4, 2, 1):
        v = op(v, v.at[lane ^ k].get(mode="promise_in_bounds"))
    return v[0]


def _chamfer_sc_body(src_hbm, tgt_hbm, srcb_hbm, tgtb_hbm, out_hbm,
                     tgt_v, t2_v, rtg_v, src_v, srcb_v):
    wid = lax.axis_index("s") * 2 + lax.axis_index("c")
    b = wid // WPB
    slot = wid % WPB

    pltpu.sync_copy(tgt_hbm.at[b], tgt_v)
    pltpu.sync_copy(tgtb_hbm.at[b], rtg_v)
    off = TC_P + slot * SPW
    for r in range(D):
        pltpu.sync_copy(src_hbm.at[b, r, pl.ds(off, SPW)], src_v.at[r])
        pltpu.sync_copy(srcb_hbm.at[b, r, pl.ds(off, SPW)], srcb_v.at[r])

    def t2_body(j, carry):
        o = j * L
        tx = tgt_v[0, pl.ds(o, L)]
        ty = tgt_v[1, pl.ds(o, L)]
        tz = tgt_v[2, pl.ds(o, L)]
        t2_v[pl.ds(o, L)] = tx * tx + ty * ty + tz * tz
        return carry

    lax.fori_loop(0, NCH, t2_body, jnp.int32(0))

    big = jnp.float32(3.0e38)
    neg2 = jnp.float32(-2.0)

    def blk_body(bi, acc):
        base = bi * L
        sxv = src_v[0, pl.ds(base, L)]
        syv = src_v[1, pl.ds(base, L)]
        szv = src_v[2, pl.ds(base, L)]
        s2v = sxv * sxv + syv * syv + szv * szv
        axv = neg2 * srcb_v[0, pl.ds(base, L)]
        ayv = neg2 * srcb_v[1, pl.ds(base, L)]
        azv = neg2 * srcb_v[2, pl.ds(base, L)]
        acc = acc + _bfly(s2v, jnp.add)

        for h in range(L // SB):
            ax = [axv[h * SB + i] for i in range(SB)]
            ay = [ayv[h * SB + i] for i in range(SB)]
            az = [azv[h * SB + i] for i in range(SB)]

            def ch_body(j, ms):
                o = j * L
                tx = rtg_v[0, pl.ds(o, L)]
                ty = rtg_v[1, pl.ds(o, L)]
                tz = rtg_v[2, pl.ds(o, L)]
                t2 = t2_v[pl.ds(o, L)]
                return tuple(
                    jnp.minimum(ms[i], t2 + ax[i] * tx + ay[i] * ty + az[i] * tz)
                    for i in range(SB)
                )

            ms0 = tuple(jnp.full((L,), big, jnp.float32) for _ in range(SB))
            ms = lax.fori_loop(0, NCH, ch_body, ms0)
            for i in range(SB):
                acc = acc + _bfly(ms[i], jnp.minimum)
        return acc

    total = lax.fori_loop(0, SPW // L, blk_body, jnp.float32(0.0))
    out_hbm_row = out_hbm.at[wid]
    res = jnp.full((L,), total, jnp.float32)

    def write_res(res_v):
        res_v[...] = res
        pltpu.sync_copy(res_v, out_hbm_row)

    pl.run_scoped(write_res, pltpu.VMEM((L,), jnp.float32))


_chamfer_sc = functools.partial(
    pl.kernel,
    out_type=jax.ShapeDtypeStruct((NW, L), jnp.float32),
    mesh=_mesh,
    scratch_types=[
        pltpu.VMEM((D, P), jnp.float32),
        pltpu.VMEM((P,), jnp.float32),
        pltpu.VMEM((D, P), jnp.float32),
        pltpu.VMEM((D, SPW), jnp.float32),
        pltpu.VMEM((D, SPW), jnp.float32),
    ],
)(_chamfer_sc_body)


def _rb16_tc(x):
    i = lax.bitcast_convert_type(x, jnp.int32)
    odd = lax.shift_right_logical(i, 16) & jnp.int32(1)
    i = (i + jnp.int32(0x7FFF) + odd) & jnp.int32(~0xFFFF)
    return lax.bitcast_convert_type(i, jnp.float32)


def _chamfer_tc_body(src_ref, tgt_ref, out_ref):
    src = src_ref[0]
    tgt = tgt_ref[0]
    t2 = jnp.sum(tgt * tgt, axis=1)
    t2a = _rb16_tc(t2)
    t2b = _rb16_tc(t2 - t2a)
    t2c = t2 - t2a - t2b
    ta = jnp.concatenate(
        [tgt, t2a[:, None], t2b[:, None], t2c[:, None]], axis=1)
    ones = jnp.ones((TB, 1), jnp.float32)
    sa = jnp.concatenate([src * jnp.float32(-2.0), ones, ones, ones], axis=1)
    dd = jax.lax.dot_general(
        ta, sa, (((1,), (1,)), ((), ())),
        preferred_element_type=jnp.float32)
    m = jnp.min(dd, axis=0)
    s2 = jnp.sum(src * src, axis=1)
    out_ref[pl.program_id(0), pl.program_id(1)] = jnp.sum(m + s2)


def _chamfer_tc(src, tgt):
    grid = (N, TC_P // TB)
    return pl.pallas_call(
        _chamfer_tc_body,
        grid=grid,
        in_specs=[
            pl.BlockSpec((1, TB, 3), lambda b, i: (b, i, 0)),
            pl.BlockSpec((1, P, 3), lambda b, i: (b, 0, 0)),
        ],
        out_specs=pl.BlockSpec(
            (N, TC_P // TB), lambda b, i: (0, 0), memory_space=pltpu.SMEM),
        out_shape=jax.ShapeDtypeStruct((N, TC_P // TB), jnp.float32),
    )(src, tgt)


def _round_bf16(x):
    i = lax.bitcast_convert_type(x, jnp.int32)
    odd = lax.shift_right_logical(i, 16) & jnp.int32(1)
    i = (i + jnp.int32(0x7FFF) + odd) & jnp.int32(~0xFFFF)
    return lax.bitcast_convert_type(i, jnp.float32)


def kernel(source_cloud, target_cloud):
    src_t = jnp.transpose(source_cloud, (0, 2, 1))
    tgt_t = jnp.transpose(target_cloud, (0, 2, 1))
    srcb_t = _round_bf16(src_t)
    tgtb_t = _round_bf16(tgt_t)
    partials = _chamfer_sc(src_t, tgt_t, srcb_t, tgtb_t)
    tc_part = _chamfer_tc(source_cloud, target_cloud)
    return (jnp.sum(partials[:, 0]) + jnp.sum(tc_part)) / N


# speedup vs baseline: 6.7366x; 1.5812x over previous
import functools

import jax
import jax.numpy as jnp
from jax import lax
from jax.experimental import pallas as pl
from jax.experimental.pallas import tpu as pltpu
from jax.experimental.pallas import tpu_sc as plsc

N, P, D = 4, 4096, 3
L = 16
NW = 32
WPB = NW // N
NCH = P // L
SC_P = 512
TC_P = P - SC_P
TB = 512
SPW = SC_P // WPB
assert SC_P % (WPB * L) == 0 and TC_P % TB == 0
SB = 8

_mesh = plsc.VectorSubcoreMesh(
    core_axis_name="c", subcore_axis_name="s", num_cores=2, num_subcores=16
)


def _bfly(v, op):
    lane = lax.iota(jnp.int32, L)
    for k in (8, 4, 2, 1):
        v = op(v, v.at[lane ^ k].get(mode="promise_in_bounds"))
    return v[0]


def _chamfer_sc_body(src_hbm, tgt_hbm, srcb_hbm, tgtb_hbm, out_hbm,
                     tgt_v, t2_v, rtg_v, src_v, srcb_v):
    wid = lax.axis_index("s") * 2 + lax.axis_index("c")
    b = wid // WPB
    slot = wid % WPB

    pltpu.sync_copy(tgt_hbm.at[b], tgt_v)
    pltpu.sync_copy(tgtb_hbm.at[b], rtg_v)
    off = TC_P + slot * SPW
    for r in range(D):
        pltpu.sync_copy(src_hbm.at[b, r, pl.ds(off, SPW)], src_v.at[r])
        pltpu.sync_copy(srcb_hbm.at[b, r, pl.ds(off, SPW)], srcb_v.at[r])

    def t2_body(j, carry):
        o = j * L
        tx = tgt_v[0, pl.ds(o, L)]
        ty = tgt_v[1, pl.ds(o, L)]
        tz = tgt_v[2, pl.ds(o, L)]
        t2_v[pl.ds(o, L)] = tx * tx + ty * ty + tz * tz
        return carry

    lax.fori_loop(0, NCH, t2_body, jnp.int32(0))

    big = jnp.float32(3.0e38)
    neg2 = jnp.float32(-2.0)

    def blk_body(bi, acc):
        base = bi * L
        sxv = src_v[0, pl.ds(base, L)]
        syv = src_v[1, pl.ds(base, L)]
        szv = src_v[2, pl.ds(base, L)]
        s2v = sxv * sxv + syv * syv + szv * szv
        axv = neg2 * srcb_v[0, pl.ds(base, L)]
        ayv = neg2 * srcb_v[1, pl.ds(base, L)]
        azv = neg2 * srcb_v[2, pl.ds(base, L)]
        acc = acc + _bfly(s2v, jnp.add)

        for h in range(L // SB):
            ax = [axv[h * SB + i] for i in range(SB)]
            ay = [ayv[h * SB + i] for i in range(SB)]
            az = [azv[h * SB + i] for i in range(SB)]

            def ch_body(j, ms):
                o = j * L
                tx = rtg_v[0, pl.ds(o, L)]
                ty = rtg_v[1, pl.ds(o, L)]
                tz = rtg_v[2, pl.ds(o, L)]
                t2 = t2_v[pl.ds(o, L)]
                return tuple(
                    jnp.minimum(ms[i], t2 + ax[i] * tx + ay[i] * ty + az[i] * tz)
                    for i in range(SB)
                )

            ms0 = tuple(jnp.full((L,), big, jnp.float32) for _ in range(SB))
            ms = lax.fori_loop(0, NCH, ch_body, ms0)
            for i in range(SB):
                acc = acc + _bfly(ms[i], jnp.minimum)
        return acc

    total = lax.fori_loop(0, SPW // L, blk_body, jnp.float32(0.0))
    out_hbm_row = out_hbm.at[wid]
    res = jnp.full((L,), total, jnp.float32)

    def write_res(res_v):
        res_v[...] = res
        pltpu.sync_copy(res_v, out_hbm_row)

    pl.run_scoped(write_res, pltpu.VMEM((L,), jnp.float32))


_chamfer_sc = functools.partial(
    pl.kernel,
    out_type=jax.ShapeDtypeStruct((NW, L), jnp.float32),
    mesh=_mesh,
    scratch_types=[
        pltpu.VMEM((D, P), jnp.float32),
        pltpu.VMEM((P,), jnp.float32),
        pltpu.VMEM((D, P), jnp.float32),
        pltpu.VMEM((D, SPW), jnp.float32),
        pltpu.VMEM((D, SPW), jnp.float32),
    ],
)(_chamfer_sc_body)


def _rb16_tc(x):
    i = lax.bitcast_convert_type(x, jnp.int32)
    odd = lax.shift_right_logical(i, 16) & jnp.int32(1)
    i = (i + jnp.int32(0x7FFF) + odd) & jnp.int32(~0xFFFF)
    return lax.bitcast_convert_type(i, jnp.float32)


def _chamfer_tc_body(src_ref, tgt_ref, out_ref):
    src = src_ref[0]
    tgt = tgt_ref[0]
    t2 = jnp.sum(tgt * tgt, axis=0)
    t2a = _rb16_tc(t2)
    t2b = _rb16_tc(t2 - t2a)
    t2c = t2 - t2a - t2b
    ta = jnp.concatenate(
        [tgt, t2a[None, :], t2b[None, :], t2c[None, :]], axis=0)
    saf = jnp.concatenate(
        [src * jnp.float32(-2.0), jnp.ones((3, P), jnp.float32)], axis=0)
    s2 = jnp.sum(src * src, axis=0)
    acc = jnp.float32(0.0)
    for i in range(TC_P // TB):
        sa = saf[:, i * TB:(i + 1) * TB]
        dd = jax.lax.dot_general(
            ta, sa, (((0,), (0,)), ((), ())),
            preferred_element_type=jnp.float32)
        m = jnp.min(dd, axis=0)
        acc = acc + jnp.sum(m + s2[i * TB:(i + 1) * TB])
    out_ref[pl.program_id(0)] = acc


def _chamfer_tc(src_t, tgt_t):
    return pl.pallas_call(
        _chamfer_tc_body,
        grid=(N,),
        in_specs=[
            pl.BlockSpec((1, D, P), lambda b: (b, 0, 0)),
            pl.BlockSpec((1, D, P), lambda b: (b, 0, 0)),
        ],
        out_specs=pl.BlockSpec(
            (N,), lambda b: (0,), memory_space=pltpu.SMEM),
        out_shape=jax.ShapeDtypeStruct((N,), jnp.float32),
    )(src_t, tgt_t)


def _round_bf16(x):
    i = lax.bitcast_convert_type(x, jnp.int32)
    odd = lax.shift_right_logical(i, 16) & jnp.int32(1)
    i = (i + jnp.int32(0x7FFF) + odd) & jnp.int32(~0xFFFF)
    return lax.bitcast_convert_type(i, jnp.float32)


def kernel(source_cloud, target_cloud):
    src_t = jnp.transpose(source_cloud, (0, 2, 1))
    tgt_t = jnp.transpose(target_cloud, (0, 2, 1))
    srcb_t = _round_bf16(src_t)
    tgtb_t = _round_bf16(tgt_t)
    partials = _chamfer_sc(src_t, tgt_t, srcb_t, tgtb_t)
    tc_part = _chamfer_tc(src_t, tgt_t)
    return (jnp.sum(partials[:, 0]) + jnp.sum(tc_part)) / N


# speedup vs baseline: 6.8189x; 1.0122x over previous
import functools

import jax
import jax.numpy as jnp
from jax import lax
from jax.experimental import pallas as pl
from jax.experimental.pallas import tpu as pltpu
from jax.experimental.pallas import tpu_sc as plsc

N, P, D = 4, 4096, 3
L = 16
NW = 32
WPB = NW // N
NCH = P // L
SC_P = 512
TC_P = P - SC_P
TB = 1792
SPW = SC_P // WPB
assert SC_P % (WPB * L) == 0 and TC_P % TB == 0
SB = 8

_mesh = plsc.VectorSubcoreMesh(
    core_axis_name="c", subcore_axis_name="s", num_cores=2, num_subcores=16
)


def _bfly(v, op):
    lane = lax.iota(jnp.int32, L)
    for k in (8, 4, 2, 1):
        v = op(v, v.at[lane ^ k].get(mode="promise_in_bounds"))
    return v[0]


def _chamfer_sc_body(src_hbm, tgt_hbm, srcb_hbm, tgtb_hbm, out_hbm,
                     tgt_v, t2_v, rtg_v, src_v, srcb_v):
    wid = lax.axis_index("s") * 2 + lax.axis_index("c")
    b = wid // WPB
    slot = wid % WPB

    pltpu.sync_copy(tgt_hbm.at[b], tgt_v)
    pltpu.sync_copy(tgtb_hbm.at[b], rtg_v)
    off = TC_P + slot * SPW
    for r in range(D):
        pltpu.sync_copy(src_hbm.at[b, r, pl.ds(off, SPW)], src_v.at[r])
        pltpu.sync_copy(srcb_hbm.at[b, r, pl.ds(off, SPW)], srcb_v.at[r])

    def t2_body(j, carry):
        o = j * L
        tx = tgt_v[0, pl.ds(o, L)]
        ty = tgt_v[1, pl.ds(o, L)]
        tz = tgt_v[2, pl.ds(o, L)]
        t2_v[pl.ds(o, L)] = tx * tx + ty * ty + tz * tz
        return carry

    lax.fori_loop(0, NCH, t2_body, jnp.int32(0))

    big = jnp.float32(3.0e38)
    neg2 = jnp.float32(-2.0)

    def blk_body(bi, acc):
        base = bi * L
        sxv = src_v[0, pl.ds(base, L)]
        syv = src_v[1, pl.ds(base, L)]
        szv = src_v[2, pl.ds(base, L)]
        s2v = sxv * sxv + syv * syv + szv * szv
        axv = neg2 * srcb_v[0, pl.ds(base, L)]
        ayv = neg2 * srcb_v[1, pl.ds(base, L)]
        azv = neg2 * srcb_v[2, pl.ds(base, L)]
        acc = acc + _bfly(s2v, jnp.add)

        for h in range(L // SB):
            ax = [axv[h * SB + i] for i in range(SB)]
            ay = [ayv[h * SB + i] for i in range(SB)]
            az = [azv[h * SB + i] for i in range(SB)]

            def ch_body(j, ms):
                o = j * L
                tx = rtg_v[0, pl.ds(o, L)]
                ty = rtg_v[1, pl.ds(o, L)]
                tz = rtg_v[2, pl.ds(o, L)]
                t2 = t2_v[pl.ds(o, L)]
                return tuple(
                    jnp.minimum(ms[i], t2 + ax[i] * tx + ay[i] * ty + az[i] * tz)
                    for i in range(SB)
                )

            ms0 = tuple(jnp.full((L,), big, jnp.float32) for _ in range(SB))
            ms = lax.fori_loop(0, NCH, ch_body, ms0)
            for i in range(SB):
                acc = acc + _bfly(ms[i], jnp.minimum)
        return acc

    total = lax.fori_loop(0, SPW // L, blk_body, jnp.float32(0.0))
    out_hbm_row = out_hbm.at[wid]
    res = jnp.full((L,), total, jnp.float32)

    def write_res(res_v):
        res_v[...] = res
        pltpu.sync_copy(res_v, out_hbm_row)

    pl.run_scoped(write_res, pltpu.VMEM((L,), jnp.float32))


_chamfer_sc = functools.partial(
    pl.kernel,
    out_type=jax.ShapeDtypeStruct((NW, L), jnp.float32),
    mesh=_mesh,
    scratch_types=[
        pltpu.VMEM((D, P), jnp.float32),
        pltpu.VMEM((P,), jnp.float32),
        pltpu.VMEM((D, P), jnp.float32),
        pltpu.VMEM((D, SPW), jnp.float32),
        pltpu.VMEM((D, SPW), jnp.float32),
    ],
)(_chamfer_sc_body)


def _rb16_tc(x):
    i = lax.bitcast_convert_type(x, jnp.int32)
    odd = lax.shift_right_logical(i, 16) & jnp.int32(1)
    i = (i + jnp.int32(0x7FFF) + odd) & jnp.int32(~0xFFFF)
    return lax.bitcast_convert_type(i, jnp.float32)


def _chamfer_tc_body(src_ref, tgt_ref, out_ref):
    for b in range(N):
        src = src_ref[b]
        tgt = tgt_ref[b]
        t2 = jnp.sum(tgt * tgt, axis=0, keepdims=True)
        t2a = _rb16_tc(t2)
        t2b = _rb16_tc(t2 - t2a)
        t2c = t2 - t2a - t2b
        ta = jnp.concatenate([tgt, t2a, t2b, t2c], axis=0)
        saf = jnp.concatenate(
            [src * jnp.float32(-2.0), jnp.ones((3, P), jnp.float32)], axis=0)
        s2 = jnp.sum(src * src, axis=0, keepdims=True)
        acc = jnp.float32(0.0)
        for i in range(TC_P // TB):
            sa = saf[:, i * TB:(i + 1) * TB]
            dd = jax.lax.dot_general(
                ta, sa, (((0,), (0,)), ((), ())),
                preferred_element_type=jnp.float32)
            m = jnp.min(dd, axis=0, keepdims=True)
            acc = acc + jnp.sum(m) + jnp.sum(s2[:, i * TB:(i + 1) * TB])
        out_ref[b] = acc


def _chamfer_tc(src_t, tgt_t):
    return pl.pallas_call(
        _chamfer_tc_body,
        in_specs=[
            pl.BlockSpec((N, D, P), lambda: (0, 0, 0)),
            pl.BlockSpec((N, D, P), lambda: (0, 0, 0)),
        ],
        out_specs=pl.BlockSpec((N,), lambda: (0,), memory_space=pltpu.SMEM),
        out_shape=jax.ShapeDtypeStruct((N,), jnp.float32),
    )(src_t, tgt_t)


def _round_bf16(x):
    i = lax.bitcast_convert_type(x, jnp.int32)
    odd = lax.shift_right_logical(i, 16) & jnp.int32(1)
    i = (i + jnp.int32(0x7FFF) + odd) & jnp.int32(~0xFFFF)
    return lax.bitcast_convert_type(i, jnp.float32)


def kernel(source_cloud, target_cloud):
    src_t = jnp.transpose(source_cloud, (0, 2, 1))
    tgt_t = jnp.transpose(target_cloud, (0, 2, 1))
    srcb_t = _round_bf16(src_t)
    tgtb_t = _round_bf16(tgt_t)
    partials = _chamfer_sc(src_t, tgt_t, srcb_t, tgtb_t)
    tc_part = _chamfer_tc(src_t, tgt_t)
    return (jnp.sum(partials[:, 0]) + jnp.sum(tc_part)) / N
